# Initial kernel scaffold; baseline (speedup 1.0000x reference)
import jax
import jax.numpy as jnp
from jax.experimental import pallas as pl


def kernel(x, edge_index, W1, b1, W2, b2):
    raise NotImplementedError("write your pallas kernel here")


# speedup vs baseline: 11.1610x; 11.1610x over previous
import functools

import jax
import jax.numpy as jnp
from jax import lax
from jax.experimental import pallas as pl
from jax.experimental.pallas import tpu as pltpu
from jax.experimental.pallas import tpu_sc as plsc

N = 10000
NP = 10240
E = 320000
D = 128
NC = 2
NS = 16
NW = NC * NS
CHUNK = 80
EDGES_PER_TILE = E // NW
NCHUNK = EDGES_PER_TILE // CHUNK
ROWS_PER_TILE = NP // NS

_mesh = plsc.VectorSubcoreMesh(
    core_axis_name="c", subcore_axis_name="s", num_cores=NC, num_subcores=NS
)


def _deg_body(ei_hbm, zeros_hbm, ones_hbm, out_hbm, idx_v, ones_v, acc_sh):
    cid = lax.axis_index("c")
    sid = lax.axis_index("s")
    wid = cid * NS + sid

    pltpu.sync_copy(ones_hbm, ones_v)

    row0 = sid * ROWS_PER_TILE
    pltpu.sync_copy(
        zeros_hbm.at[pl.ds(row0, ROWS_PER_TILE)],
        acc_sh.at[pl.ds(row0, ROWS_PER_TILE)],
    )
    plsc.subcore_barrier()

    base_ck = wid * NCHUNK

    @pl.loop(0, NCHUNK)
    def _(j):
        pltpu.sync_copy(ei_hbm.at[base_ck + j], idx_v)
        pltpu.sync_copy(ones_v, acc_sh.at[idx_v.at[1]], add=True)

    plsc.subcore_barrier()
    pltpu.sync_copy(
        acc_sh.at[pl.ds(row0, ROWS_PER_TILE)],
        out_hbm.at[cid, pl.ds(row0, ROWS_PER_TILE)],
    )


@jax.jit
def _deg_partials(ei_r, zeros128, ones128):
    return pl.kernel(
        _deg_body,
        out_type=jax.ShapeDtypeStruct((NC, NP, D), jnp.float32),
        mesh=_mesh,
        scratch_types=[
            pltpu.VMEM((2, CHUNK), jnp.int32),
            pltpu.VMEM((CHUNK, D), jnp.float32),
            pltpu.VMEM_SHARED((NP, D), jnp.float32),
        ],
    )(ei_r, zeros128, ones128)


def _agg_body(y_hbm, ei_hbm, zeros_hbm, out_hbm, idx_v, rows_v, acc_sh):
    cid = lax.axis_index("c")
    sid = lax.axis_index("s")
    wid = cid * NS + sid

    row0 = sid * ROWS_PER_TILE
    pltpu.sync_copy(
        zeros_hbm.at[pl.ds(row0, ROWS_PER_TILE)],
        acc_sh.at[pl.ds(row0, ROWS_PER_TILE)],
    )
    plsc.subcore_barrier()

    base_ck = wid * NCHUNK

    @pl.loop(0, NCHUNK)
    def _(j):
        pltpu.sync_copy(ei_hbm.at[base_ck + j], idx_v)
        pltpu.sync_copy(y_hbm.at[idx_v.at[0]], rows_v)
        pltpu.sync_copy(rows_v, acc_sh.at[idx_v.at[1]], add=True)

    plsc.subcore_barrier()
    pltpu.sync_copy(
        acc_sh.at[pl.ds(row0, ROWS_PER_TILE)],
        out_hbm.at[cid, pl.ds(row0, ROWS_PER_TILE)],
    )


@jax.jit
def _aggregate(y, ei_r, zeros128):
    return pl.kernel(
        _agg_body,
        out_type=jax.ShapeDtypeStruct((NC, NP, D), jnp.float32),
        mesh=_mesh,
        scratch_types=[
            pltpu.VMEM((2, CHUNK), jnp.int32),
            pltpu.VMEM((CHUNK, D), jnp.float32),
            pltpu.VMEM_SHARED((NP, D), jnp.float32),
        ],
    )(y, ei_r, zeros128)


_MB = 1000


def _mm_scale_body(x_ref, wt_ref, dis_ref, o_ref):
    acc = jnp.dot(x_ref[...], wt_ref[...], preferred_element_type=jnp.float32)
    o_ref[...] = acc * dis_ref[...]


@jax.jit
def _mm_scale(x, wt, dis):
    return pl.pallas_call(
        _mm_scale_body,
        grid=(N // _MB,),
        in_specs=[
            pl.BlockSpec((_MB, D), lambda i: (i, 0)),
            pl.BlockSpec((D, D), lambda i: (0, 0)),
            pl.BlockSpec((_MB, 1), lambda i: (i, 0)),
        ],
        out_specs=pl.BlockSpec((_MB, D), lambda i: (i, 0)),
        out_shape=jax.ShapeDtypeStruct((N, D), jnp.float32),
    )(x, wt, dis)


def _mid_body(p_ref, dis_ref, b_ref, wt_ref, o_ref):
    h = dis_ref[...] * (p_ref[0] + p_ref[1]) + b_ref[...]
    h = jnp.maximum(h, 0.0)
    acc = jnp.dot(h, wt_ref[...], preferred_element_type=jnp.float32)
    o_ref[...] = acc * dis_ref[...]


@jax.jit
def _mid_layer(p, dis, b, wt):
    return pl.pallas_call(
        _mid_body,
        grid=(N // _MB,),
        in_specs=[
            pl.BlockSpec((NC, _MB, D), lambda i: (0, i, 0)),
            pl.BlockSpec((_MB, 1), lambda i: (i, 0)),
            pl.BlockSpec((1, D), lambda i: (0, 0)),
            pl.BlockSpec((D, D), lambda i: (0, 0)),
        ],
        out_specs=pl.BlockSpec((_MB, D), lambda i: (i, 0)),
        out_shape=jax.ShapeDtypeStruct((N, D), jnp.float32),
    )(p, dis, b, wt)


def _final_body(q_ref, dis_ref, b_ref, o_ref):
    o_ref[...] = dis_ref[...] * (q_ref[0] + q_ref[1]) + b_ref[...]


@jax.jit
def _final_layer(q, dis, b):
    return pl.pallas_call(
        _final_body,
        grid=(N // _MB,),
        in_specs=[
            pl.BlockSpec((NC, _MB, D), lambda i: (0, i, 0)),
            pl.BlockSpec((_MB, 1), lambda i: (i, 0)),
            pl.BlockSpec((1, D), lambda i: (0, 0)),
        ],
        out_specs=pl.BlockSpec((_MB, D), lambda i: (i, 0)),
        out_shape=jax.ShapeDtypeStruct((N, D), jnp.float32),
    )(q, dis, b)


def kernel(x, edge_index, W1, b1, W2, b2):
    ei = edge_index.astype(jnp.int32)
    ei_r = ei.reshape(2, E // CHUNK, CHUNK).transpose(1, 0, 2)

    zeros128 = jnp.zeros((NP, D), jnp.float32)

    ones128 = jnp.ones((CHUNK, D), jnp.float32)
    degp = _deg_partials(ei_r, zeros128, ones128)
    deg = degp[0, :N, 0] + degp[1, :N, 0]
    dis = jnp.where(deg > 0, lax.rsqrt(deg), 0.0).reshape(N, 1)

    y1 = _mm_scale(x, W1.T, dis)
    p = _aggregate(y1, ei_r, zeros128)
    y2 = _mid_layer(p, dis, b1.reshape(1, D), W2.T)
    q = _aggregate(y2, ei_r, zeros128)
    out = _final_layer(q, dis, b2.reshape(1, D))
    return out


# speedup vs baseline: 16.7632x; 1.5019x over previous
import functools

import jax
import jax.numpy as jnp
from jax import lax
from jax.experimental import pallas as pl
from jax.experimental.pallas import tpu as pltpu
from jax.experimental.pallas import tpu_sc as plsc

N = 10000
NP = 10240
E = 320000
D = 128
NC = 2
NS = 16
NW = NC * NS
CHUNK = 80
EDGES_PER_TILE = E // NW
NCHUNK = EDGES_PER_TILE // CHUNK
G = 5
NG = NCHUNK // G
GA = 4
NGA = NCHUNK // GA
ROWS_PER_TILE = NP // NS

_mesh = plsc.VectorSubcoreMesh(
    core_axis_name="c", subcore_axis_name="s", num_cores=NC, num_subcores=NS
)


def _deg_body(ei_hbm, zeros_hbm, ones_hbm, out_hbm, idx_v, ones_v, acc_sh, sem):
    cid = lax.axis_index("c")
    sid = lax.axis_index("s")
    wid = cid * NS + sid

    pltpu.sync_copy(ones_hbm, ones_v)

    row0 = sid * ROWS_PER_TILE
    pltpu.sync_copy(
        zeros_hbm.at[pl.ds(row0, ROWS_PER_TILE)],
        acc_sh.at[pl.ds(row0, ROWS_PER_TILE)],
    )
    plsc.subcore_barrier()

    base_ck = wid * NCHUNK

    @pl.loop(0, NG)
    def _(g):
        pltpu.sync_copy(ei_hbm.at[pl.ds(base_ck + g * G, G)], idx_v)
        descs = [
            pltpu.async_copy(ones_v, acc_sh.at[idx_v.at[b, 1]], sem, add=True)
            for b in range(G)
        ]
        for d_ in descs:
            d_.wait()

    plsc.subcore_barrier()
    pltpu.sync_copy(
        acc_sh.at[pl.ds(row0, ROWS_PER_TILE)],
        out_hbm.at[cid, pl.ds(row0, ROWS_PER_TILE)],
    )


@jax.jit
def _deg_partials(ei_r, zeros128, ones128):
    return pl.kernel(
        _deg_body,
        out_type=jax.ShapeDtypeStruct((NC, NP, D), jnp.float32),
        mesh=_mesh,
        scratch_types=[
            pltpu.VMEM((G, 2, CHUNK), jnp.int32),
            pltpu.VMEM((CHUNK, D), jnp.float32),
            pltpu.VMEM_SHARED((NP, D), jnp.float32),
            pltpu.SemaphoreType.DMA,
        ],
    )(ei_r, zeros128, ones128)


def _agg_body(y_hbm, ei_hbm, zeros_hbm, out_hbm, idx_v, rows_v, acc_sh, sem):
    cid = lax.axis_index("c")
    sid = lax.axis_index("s")
    wid = cid * NS + sid

    row0 = sid * ROWS_PER_TILE
    pltpu.sync_copy(
        zeros_hbm.at[pl.ds(row0, ROWS_PER_TILE)],
        acc_sh.at[pl.ds(row0, ROWS_PER_TILE)],
    )
    plsc.subcore_barrier()

    base_ck = wid * NCHUNK

    @pl.loop(0, NGA)
    def _(g):
        pltpu.sync_copy(ei_hbm.at[pl.ds(base_ck + g * GA, GA)], idx_v)
        descs = [
            pltpu.async_copy(y_hbm.at[idx_v.at[b, 0]], rows_v.at[b], sem)
            for b in range(GA)
        ]
        for d_ in descs:
            d_.wait()
        for b in range(GA):
            pltpu.sync_copy(rows_v.at[b], acc_sh.at[idx_v.at[b, 1]], add=True)

    pltpu.sync_copy(ei_hbm.at[pl.ds(base_ck + GA * NGA, 1)], idx_v.at[pl.ds(0, 1)])
    pltpu.sync_copy(y_hbm.at[idx_v.at[0, 0]], rows_v.at[0])
    pltpu.sync_copy(rows_v.at[0], acc_sh.at[idx_v.at[0, 1]], add=True)

    plsc.subcore_barrier()
    pltpu.sync_copy(
        acc_sh.at[pl.ds(row0, ROWS_PER_TILE)],
        out_hbm.at[cid, pl.ds(row0, ROWS_PER_TILE)],
    )


@jax.jit
def _aggregate(y, ei_r, zeros128):
    return pl.kernel(
        _agg_body,
        out_type=jax.ShapeDtypeStruct((NC, NP, D), jnp.float32),
        mesh=_mesh,
        scratch_types=[
            pltpu.VMEM((GA, 2, CHUNK), jnp.int32),
            pltpu.VMEM((GA, CHUNK, D), jnp.float32),
            pltpu.VMEM_SHARED((NP, D), jnp.float32),
            pltpu.SemaphoreType.DMA,
        ],
    )(y, ei_r, zeros128)


_MB = 1000


def _mm_scale_body(x_ref, wt_ref, dis_ref, o_ref):
    acc = jnp.dot(x_ref[...], wt_ref[...], preferred_element_type=jnp.float32)
    o_ref[...] = acc * dis_ref[...]


@jax.jit
def _mm_scale(x, wt, dis):
    return pl.pallas_call(
        _mm_scale_body,
        grid=(N // _MB,),
        in_specs=[
            pl.BlockSpec((_MB, D), lambda i: (i, 0)),
            pl.BlockSpec((D, D), lambda i: (0, 0)),
            pl.BlockSpec((_MB, 1), lambda i: (i, 0)),
        ],
        out_specs=pl.BlockSpec((_MB, D), lambda i: (i, 0)),
        out_shape=jax.ShapeDtypeStruct((N, D), jnp.float32),
    )(x, wt, dis)


def _mid_body(p_ref, dis_ref, b_ref, wt_ref, o_ref):
    h = dis_ref[...] * (p_ref[0] + p_ref[1]) + b_ref[...]
    h = jnp.maximum(h, 0.0)
    acc = jnp.dot(h, wt_ref[...], preferred_element_type=jnp.float32)
    o_ref[...] = acc * dis_ref[...]


@jax.jit
def _mid_layer(p, dis, b, wt):
    return pl.pallas_call(
        _mid_body,
        grid=(N // _MB,),
        in_specs=[
            pl.BlockSpec((NC, _MB, D), lambda i: (0, i, 0)),
            pl.BlockSpec((_MB, 1), lambda i: (i, 0)),
            pl.BlockSpec((1, D), lambda i: (0, 0)),
            pl.BlockSpec((D, D), lambda i: (0, 0)),
        ],
        out_specs=pl.BlockSpec((_MB, D), lambda i: (i, 0)),
        out_shape=jax.ShapeDtypeStruct((N, D), jnp.float32),
    )(p, dis, b, wt)


def _final_body(q_ref, dis_ref, b_ref, o_ref):
    o_ref[...] = dis_ref[...] * (q_ref[0] + q_ref[1]) + b_ref[...]


@jax.jit
def _final_layer(q, dis, b):
    return pl.pallas_call(
        _final_body,
        grid=(N // _MB,),
        in_specs=[
            pl.BlockSpec((NC, _MB, D), lambda i: (0, i, 0)),
            pl.BlockSpec((_MB, 1), lambda i: (i, 0)),
            pl.BlockSpec((1, D), lambda i: (0, 0)),
        ],
        out_specs=pl.BlockSpec((_MB, D), lambda i: (i, 0)),
        out_shape=jax.ShapeDtypeStruct((N, D), jnp.float32),
    )(q, dis, b)


def kernel(x, edge_index, W1, b1, W2, b2):
    ei = edge_index.astype(jnp.int32)
    ei_r = ei.reshape(2, E // CHUNK, CHUNK).transpose(1, 0, 2)

    zeros128 = jnp.zeros((NP, D), jnp.float32)

    ones128 = jnp.ones((CHUNK, D), jnp.float32)
    degp = _deg_partials(ei_r, zeros128, ones128)
    deg = degp[0, :N, 0] + degp[1, :N, 0]
    dis = jnp.where(deg > 0, lax.rsqrt(deg), 0.0).reshape(N, 1)

    y1 = _mm_scale(x, W1.T, dis)
    p = _aggregate(y1, ei_r, zeros128)
    y2 = _mid_layer(p, dis, b1.reshape(1, D), W2.T)
    q = _aggregate(y2, ei_r, zeros128)
    out = _final_layer(q, dis, b2.reshape(1, D))
    return out


# speedup vs baseline: 21.0392x; 1.2551x over previous
import functools

import jax
import jax.numpy as jnp
from jax import lax
from jax.experimental import pallas as pl
from jax.experimental.pallas import tpu as pltpu
from jax.experimental.pallas import tpu_sc as plsc

N = 10000
NP = 10240
E = 320000
D = 128
NC = 2
NS = 16
NW = NC * NS
CHUNK = 80
EDGES_PER_TILE = E // NW
NCHUNK = EDGES_PER_TILE // CHUNK
G = 5
NG = NCHUNK // G
GA = 4
NGA = NCHUNK // GA
ROWS_PER_TILE = NP // NS

_mesh = plsc.VectorSubcoreMesh(
    core_axis_name="c", subcore_axis_name="s", num_cores=NC, num_subcores=NS
)


def _deg_body(ei_hbm, zeros_hbm, ones_hbm, out_hbm, idx_v, ones_v, acc_sh, sem):
    cid = lax.axis_index("c")
    sid = lax.axis_index("s")
    wid = cid * NS + sid

    pltpu.sync_copy(ones_hbm, ones_v)

    row0 = sid * ROWS_PER_TILE
    pltpu.sync_copy(
        zeros_hbm.at[pl.ds(row0, ROWS_PER_TILE)],
        acc_sh.at[pl.ds(row0, ROWS_PER_TILE)],
    )
    plsc.subcore_barrier()

    base_ck = wid * NCHUNK

    @pl.loop(0, NG)
    def _(g):
        pltpu.sync_copy(ei_hbm.at[pl.ds(base_ck + g * G, G)], idx_v)
        descs = [
            pltpu.async_copy(ones_v, acc_sh.at[idx_v.at[b, 1]], sem, add=True)
            for b in range(G)
        ]
        for d_ in descs:
            d_.wait()

    plsc.subcore_barrier()
    pltpu.sync_copy(
        acc_sh.at[pl.ds(row0, ROWS_PER_TILE)],
        out_hbm.at[cid, pl.ds(row0, ROWS_PER_TILE)],
    )


@jax.jit
def _deg_partials(ei_r, zeros128, ones128):
    return pl.kernel(
        _deg_body,
        out_type=jax.ShapeDtypeStruct((NC, NP, D), jnp.float32),
        mesh=_mesh,
        scratch_types=[
            pltpu.VMEM((G, 2, CHUNK), jnp.int32),
            pltpu.VMEM((CHUNK, D), jnp.float32),
            pltpu.VMEM_SHARED((NP, D), jnp.float32),
            pltpu.SemaphoreType.DMA,
        ],
    )(ei_r, zeros128, ones128)


G2 = 2
NPAIR = 31


def _agg_body(y_hbm, ei_hbm, zeros_hbm, out_hbm, idxa, idxb, rowsa, rowsb,
              acc_sh, gsa, gsb, ssa, ssb):
    cid = lax.axis_index("c")
    sid = lax.axis_index("s")
    wid = cid * NS + sid

    row0 = sid * ROWS_PER_TILE
    pltpu.sync_copy(
        zeros_hbm.at[pl.ds(row0, ROWS_PER_TILE)],
        acc_sh.at[pl.ds(row0, ROWS_PER_TILE)],
    )
    plsc.subcore_barrier()

    base_ck = wid * NCHUNK

    def fire(idx_v, rows_v, gsem, ck):
        pltpu.sync_copy(ei_hbm.at[pl.ds(ck, G2)], idx_v)
        for b in range(G2):
            pltpu.async_copy(y_hbm.at[idx_v.at[b, 0]], rows_v.at[b], gsem)

    def drain(sem, rows_v):
        for b in range(G2):
            pltpu.make_async_copy(y_hbm.at[pl.ds(0, CHUNK)], rows_v.at[b], sem).wait()

    def scatter(idx_v, rows_v, ssem):
        for b in range(G2):
            pltpu.async_copy(rows_v.at[b], acc_sh.at[idx_v.at[b, 1]], ssem, add=True)

    fire(idxa, rowsa, gsa, base_ck)

    @pl.loop(0, NPAIR)
    def _(t):
        fire(idxb, rowsb, gsb, base_ck + (2 * t + 1) * G2)
        drain(gsa, rowsa)
        scatter(idxa, rowsa, ssa)

        @pl.when(t < NPAIR - 1)
        def _():
            drain(ssa, rowsa)
            fire(idxa, rowsa, gsa, base_ck + (2 * t + 2) * G2)

        drain(gsb, rowsb)
        scatter(idxb, rowsb, ssb)
        drain(ssb, rowsb)

    drain(ssa, rowsa)
    pltpu.sync_copy(ei_hbm.at[pl.ds(base_ck + 2 * NPAIR * G2, 1)], idxa.at[pl.ds(0, 1)])
    pltpu.sync_copy(y_hbm.at[idxa.at[0, 0]], rowsa.at[0])
    pltpu.sync_copy(rowsa.at[0], acc_sh.at[idxa.at[0, 1]], add=True)

    plsc.subcore_barrier()
    pltpu.sync_copy(
        acc_sh.at[pl.ds(row0, ROWS_PER_TILE)],
        out_hbm.at[cid, pl.ds(row0, ROWS_PER_TILE)],
    )


@jax.jit
def _aggregate(y, ei_r, zeros128):
    return pl.kernel(
        _agg_body,
        out_type=jax.ShapeDtypeStruct((NC, NP, D), jnp.float32),
        mesh=_mesh,
        scratch_types=[
            pltpu.VMEM((G2, 2, CHUNK), jnp.int32),
            pltpu.VMEM((G2, 2, CHUNK), jnp.int32),
            pltpu.VMEM((G2, CHUNK, D), jnp.float32),
            pltpu.VMEM((G2, CHUNK, D), jnp.float32),
            pltpu.VMEM_SHARED((NP, D), jnp.float32),
            pltpu.SemaphoreType.DMA,
            pltpu.SemaphoreType.DMA,
            pltpu.SemaphoreType.DMA,
            pltpu.SemaphoreType.DMA,
        ],
    )(y, ei_r, zeros128)


_MB = 1000


def _mm_scale_body(x_ref, wt_ref, dis_ref, o_ref):
    acc = jnp.dot(x_ref[...], wt_ref[...], preferred_element_type=jnp.float32)
    o_ref[...] = acc * dis_ref[...]


@jax.jit
def _mm_scale(x, wt, dis):
    return pl.pallas_call(
        _mm_scale_body,
        grid=(N // _MB,),
        in_specs=[
            pl.BlockSpec((_MB, D), lambda i: (i, 0)),
            pl.BlockSpec((D, D), lambda i: (0, 0)),
            pl.BlockSpec((_MB, 1), lambda i: (i, 0)),
        ],
        out_specs=pl.BlockSpec((_MB, D), lambda i: (i, 0)),
        out_shape=jax.ShapeDtypeStruct((N, D), jnp.float32),
    )(x, wt, dis)


def _mid_body(p_ref, dis_ref, b_ref, wt_ref, o_ref):
    h = dis_ref[...] * (p_ref[0] + p_ref[1]) + b_ref[...]
    h = jnp.maximum(h, 0.0)
    acc = jnp.dot(h, wt_ref[...], preferred_element_type=jnp.float32)
    o_ref[...] = acc * dis_ref[...]


@jax.jit
def _mid_layer(p, dis, b, wt):
    return pl.pallas_call(
        _mid_body,
        grid=(N // _MB,),
        in_specs=[
            pl.BlockSpec((NC, _MB, D), lambda i: (0, i, 0)),
            pl.BlockSpec((_MB, 1), lambda i: (i, 0)),
            pl.BlockSpec((1, D), lambda i: (0, 0)),
            pl.BlockSpec((D, D), lambda i: (0, 0)),
        ],
        out_specs=pl.BlockSpec((_MB, D), lambda i: (i, 0)),
        out_shape=jax.ShapeDtypeStruct((N, D), jnp.float32),
    )(p, dis, b, wt)


def _final_body(q_ref, dis_ref, b_ref, o_ref):
    o_ref[...] = dis_ref[...] * (q_ref[0] + q_ref[1]) + b_ref[...]


@jax.jit
def _final_layer(q, dis, b):
    return pl.pallas_call(
        _final_body,
        grid=(N // _MB,),
        in_specs=[
            pl.BlockSpec((NC, _MB, D), lambda i: (0, i, 0)),
            pl.BlockSpec((_MB, 1), lambda i: (i, 0)),
            pl.BlockSpec((1, D), lambda i: (0, 0)),
        ],
        out_specs=pl.BlockSpec((_MB, D), lambda i: (i, 0)),
        out_shape=jax.ShapeDtypeStruct((N, D), jnp.float32),
    )(q, dis, b)


def kernel(x, edge_index, W1, b1, W2, b2):
    ei = edge_index.astype(jnp.int32)
    ei_r = ei.reshape(2, E // CHUNK, CHUNK).transpose(1, 0, 2)

    zeros128 = jnp.zeros((NP, D), jnp.float32)

    ones128 = jnp.ones((CHUNK, D), jnp.float32)
    degp = _deg_partials(ei_r, zeros128, ones128)
    deg = degp[0, :N, 0] + degp[1, :N, 0]
    dis = jnp.where(deg > 0, lax.rsqrt(deg), 0.0).reshape(N, 1)

    y1 = _mm_scale(x, W1.T, dis)
    p = _aggregate(y1, ei_r, zeros128)
    y2 = _mid_layer(p, dis, b1.reshape(1, D), W2.T)
    q = _aggregate(y2, ei_r, zeros128)
    out = _final_layer(q, dis, b2.reshape(1, D))
    return out


# speedup vs baseline: 21.7921x; 1.0358x over previous
import functools

import jax
import jax.numpy as jnp
from jax import lax
from jax.experimental import pallas as pl
from jax.experimental.pallas import tpu as pltpu
from jax.experimental.pallas import tpu_sc as plsc

N = 10000
NP = 10240
E = 320000
D = 128
NC = 2
NS = 16
NW = NC * NS
CHUNK = 80
EDGES_PER_TILE = E // NW
NCHUNK = EDGES_PER_TILE // CHUNK
G = 5
NG = NCHUNK // G
GA = 4
NGA = NCHUNK // GA
ROWS_PER_TILE = NP // NS

_mesh = plsc.VectorSubcoreMesh(
    core_axis_name="c", subcore_axis_name="s", num_cores=NC, num_subcores=NS
)


def _deg_body(ei_hbm, zeros_hbm, ones_hbm, out_hbm, idx_v, ones_v, acc_sh, sem):
    cid = lax.axis_index("c")
    sid = lax.axis_index("s")
    wid = cid * NS + sid

    pltpu.sync_copy(ones_hbm, ones_v)

    row0 = sid * ROWS_PER_TILE
    pltpu.sync_copy(
        zeros_hbm.at[pl.ds(row0, ROWS_PER_TILE)],
        acc_sh.at[pl.ds(row0, ROWS_PER_TILE)],
    )
    plsc.subcore_barrier()

    base_ck = wid * NCHUNK
    pltpu.sync_copy(ei_hbm.at[pl.ds(base_ck, NCHUNK)], idx_v)

    @pl.loop(0, NG)
    def _(g):
        descs = [
            pltpu.async_copy(ones_v, acc_sh.at[idx_v.at[g * G + b, 1]], sem, add=True)
            for b in range(G)
        ]
        for d_ in descs:
            d_.wait()

    plsc.subcore_barrier()
    pltpu.sync_copy(
        acc_sh.at[pl.ds(row0, ROWS_PER_TILE)],
        out_hbm.at[cid, pl.ds(row0, ROWS_PER_TILE)],
    )


@jax.jit
def _deg_partials(ei_r, zeros128, ones128):
    return pl.kernel(
        _deg_body,
        out_type=jax.ShapeDtypeStruct((NC, NP, D), jnp.float32),
        mesh=_mesh,
        scratch_types=[
            pltpu.VMEM((NCHUNK, 2, CHUNK), jnp.int32),
            pltpu.VMEM((CHUNK, D), jnp.float32),
            pltpu.VMEM_SHARED((NP, D), jnp.float32),
            pltpu.SemaphoreType.DMA,
        ],
    )(ei_r, zeros128, ones128)


G2 = 2
NPAIR = 31


def _agg_body(y_hbm, ei_hbm, zeros_hbm, out_hbm, idxa, idxb, rowsa, rowsb,
              acc_sh, gsa, gsb, ssa, ssb):
    cid = lax.axis_index("c")
    sid = lax.axis_index("s")
    wid = cid * NS + sid

    row0 = sid * ROWS_PER_TILE
    pltpu.sync_copy(
        zeros_hbm.at[pl.ds(row0, ROWS_PER_TILE)],
        acc_sh.at[pl.ds(row0, ROWS_PER_TILE)],
    )
    plsc.subcore_barrier()

    base_ck = wid * NCHUNK

    def fire(idx_v, rows_v, gsem, ck):
        pltpu.sync_copy(ei_hbm.at[pl.ds(ck, G2)], idx_v)
        for b in range(G2):
            pltpu.async_copy(y_hbm.at[idx_v.at[b, 0]], rows_v.at[b], gsem)

    def drain(sem, rows_v):
        for b in range(G2):
            pltpu.make_async_copy(y_hbm.at[pl.ds(0, CHUNK)], rows_v.at[b], sem).wait()

    def scatter(idx_v, rows_v, ssem):
        for b in range(G2):
            pltpu.async_copy(rows_v.at[b], acc_sh.at[idx_v.at[b, 1]], ssem, add=True)

    fire(idxa, rowsa, gsa, base_ck)

    @pl.loop(0, NPAIR)
    def _(t):
        fire(idxb, rowsb, gsb, base_ck + (2 * t + 1) * G2)
        drain(gsa, rowsa)
        scatter(idxa, rowsa, ssa)

        @pl.when(t < NPAIR - 1)
        def _():
            drain(ssa, rowsa)
            fire(idxa, rowsa, gsa, base_ck + (2 * t + 2) * G2)

        drain(gsb, rowsb)
        scatter(idxb, rowsb, ssb)
        drain(ssb, rowsb)

    drain(ssa, rowsa)
    pltpu.sync_copy(ei_hbm.at[pl.ds(base_ck + 2 * NPAIR * G2, 1)], idxa.at[pl.ds(0, 1)])
    pltpu.sync_copy(y_hbm.at[idxa.at[0, 0]], rowsa.at[0])
    pltpu.sync_copy(rowsa.at[0], acc_sh.at[idxa.at[0, 1]], add=True)

    plsc.subcore_barrier()
    pltpu.sync_copy(
        acc_sh.at[pl.ds(row0, ROWS_PER_TILE)],
        out_hbm.at[cid, pl.ds(row0, ROWS_PER_TILE)],
    )


@jax.jit
def _aggregate(y, ei_r, zeros128):
    return pl.kernel(
        _agg_body,
        out_type=jax.ShapeDtypeStruct((NC, NP, D), jnp.float32),
        mesh=_mesh,
        scratch_types=[
            pltpu.VMEM((G2, 2, CHUNK), jnp.int32),
            pltpu.VMEM((G2, 2, CHUNK), jnp.int32),
            pltpu.VMEM((G2, CHUNK, D), jnp.float32),
            pltpu.VMEM((G2, CHUNK, D), jnp.float32),
            pltpu.VMEM_SHARED((NP, D), jnp.float32),
            pltpu.SemaphoreType.DMA,
            pltpu.SemaphoreType.DMA,
            pltpu.SemaphoreType.DMA,
            pltpu.SemaphoreType.DMA,
        ],
    )(y, ei_r, zeros128)


_MB = 1000


_DN_T = (((1,), (1,)), ((), ()))


def _mm_scale_body(x_ref, w_ref, degp_ref, o_ref, dis_ref):
    deg = degp_ref[0, :, 0:1] + degp_ref[1, :, 0:1]
    dis = jnp.where(deg > 0, lax.rsqrt(jnp.maximum(deg, 1.0)), 0.0)
    dis_ref[...] = dis
    acc = lax.dot_general(x_ref[...], w_ref[...], _DN_T,
                          preferred_element_type=jnp.float32)
    o_ref[...] = acc * dis


@jax.jit
def _mm_scale(x, w, degp):
    return pl.pallas_call(
        _mm_scale_body,
        grid=(N // _MB,),
        in_specs=[
            pl.BlockSpec((_MB, D), lambda i: (i, 0)),
            pl.BlockSpec((D, D), lambda i: (0, 0)),
            pl.BlockSpec((NC, _MB, D), lambda i: (0, i, 0)),
        ],
        out_specs=[
            pl.BlockSpec((_MB, D), lambda i: (i, 0)),
            pl.BlockSpec((_MB, 1), lambda i: (i, 0)),
        ],
        out_shape=[
            jax.ShapeDtypeStruct((N, D), jnp.float32),
            jax.ShapeDtypeStruct((N, 1), jnp.float32),
        ],
    )(x, w, degp)


def _mid_body(p_ref, dis_ref, b_ref, w_ref, o_ref):
    h = dis_ref[...] * (p_ref[0] + p_ref[1]) + b_ref[...]
    h = jnp.maximum(h, 0.0)
    acc = lax.dot_general(h, w_ref[...], _DN_T,
                          preferred_element_type=jnp.float32)
    o_ref[...] = acc * dis_ref[...]


@jax.jit
def _mid_layer(p, dis, b, w):
    return pl.pallas_call(
        _mid_body,
        grid=(N // _MB,),
        in_specs=[
            pl.BlockSpec((NC, _MB, D), lambda i: (0, i, 0)),
            pl.BlockSpec((_MB, 1), lambda i: (i, 0)),
            pl.BlockSpec((1, D), lambda i: (0, 0)),
            pl.BlockSpec((D, D), lambda i: (0, 0)),
        ],
        out_specs=pl.BlockSpec((_MB, D), lambda i: (i, 0)),
        out_shape=jax.ShapeDtypeStruct((N, D), jnp.float32),
    )(p, dis, b, w)


def _final_body(q_ref, dis_ref, b_ref, o_ref):
    o_ref[...] = dis_ref[...] * (q_ref[0] + q_ref[1]) + b_ref[...]


@jax.jit
def _final_layer(q, dis, b):
    return pl.pallas_call(
        _final_body,
        grid=(N // _MB,),
        in_specs=[
            pl.BlockSpec((NC, _MB, D), lambda i: (0, i, 0)),
            pl.BlockSpec((_MB, 1), lambda i: (i, 0)),
            pl.BlockSpec((1, D), lambda i: (0, 0)),
        ],
        out_specs=pl.BlockSpec((_MB, D), lambda i: (i, 0)),
        out_shape=jax.ShapeDtypeStruct((N, D), jnp.float32),
    )(q, dis, b)


def kernel(x, edge_index, W1, b1, W2, b2):
    ei = edge_index.astype(jnp.int32)
    ei_r = ei.reshape(2, E // CHUNK, CHUNK).transpose(1, 0, 2)

    zeros128 = jnp.zeros((NP, D), jnp.float32)

    ones128 = jnp.ones((CHUNK, D), jnp.float32)
    degp = _deg_partials(ei_r, zeros128, ones128)
    y1, dis = _mm_scale(x, W1, degp)
    p = _aggregate(y1, ei_r, zeros128)
    y2 = _mid_layer(p, dis, b1.reshape(1, D), W2)
    q = _aggregate(y2, ei_r, zeros128)
    out = _final_layer(q, dis, b2.reshape(1, D))
    return out
